# Initial kernel scaffold; baseline (speedup 1.0000x reference)
import jax
import jax.numpy as jnp
from jax.experimental import pallas as pl


def kernel(x, edge_index, W1, b1, W2, b2):
    raise NotImplementedError("write your pallas kernel here")


# speedup vs baseline: 7.2306x; 7.2306x over previous
import functools

import jax
import jax.numpy as jnp
from jax import lax
from jax.experimental import pallas as pl
from jax.experimental.pallas import tpu as pltpu
from jax.experimental.pallas import tpu_sc as plsc

N = 10000
D = 128
E = 320000

NC = 2
NS = 16
NW = NC * NS
CH = 128
CPW = 80
EPW = CPW * CH
E_PAD = NW * EPW
ROWS_PT = 640
N_PAD = NS * ROWS_PT
TRASH = N

_mesh = plsc.VectorSubcoreMesh(core_axis_name="c", subcore_axis_name="s")


def _deg_body(dst_hbm, zeros_hbm, out_hbm, dbuf, onesv, deg_sh):
    c = lax.axis_index("c")
    s = lax.axis_index("s")
    w = s * NC + c

    pltpu.sync_copy(zeros_hbm.at[pl.ds(s * ROWS_PT, ROWS_PT)],
                    deg_sh.at[pl.ds(s * ROWS_PT, ROWS_PT)])

    def _orow(i, carry):
        onesv[i, :] = jnp.ones((16,), jnp.float32)
        return carry

    lax.fori_loop(0, CH, _orow, 0)
    plsc.subcore_barrier()

    def _step(j, carry):
        pltpu.sync_copy(dst_hbm.at[w * CPW + j], dbuf)
        pltpu.sync_copy(onesv, deg_sh.at[dbuf], add=True)
        return carry

    lax.fori_loop(0, CPW, _step, 0)
    plsc.subcore_barrier()
    pltpu.sync_copy(deg_sh.at[pl.ds(s * ROWS_PT, ROWS_PT)],
                    out_hbm.at[pl.ds(c * N_PAD + s * ROWS_PT, ROWS_PT)])


_deg = functools.partial(
    pl.kernel,
    mesh=_mesh,
    out_type=jax.ShapeDtypeStruct((NC * N_PAD, 16), jnp.float32),
    scratch_types=[
        pltpu.VMEM((CH,), jnp.int32),
        pltpu.VMEM((CH, 16), jnp.float32),
        pltpu.VMEM_SHARED((N_PAD, 16), jnp.float32),
    ],
)(_deg_body)


def _agg_body(tab_hbm, src_hbm, dst_hbm, zeros_hbm, out_hbm,
              sbuf, dbuf, rowsv, acc_sh):
    c = lax.axis_index("c")
    s = lax.axis_index("s")
    w = s * NC + c

    pltpu.sync_copy(zeros_hbm.at[pl.ds(s * ROWS_PT, ROWS_PT)],
                    acc_sh.at[pl.ds(s * ROWS_PT, ROWS_PT)])
    plsc.subcore_barrier()

    def _step(j, carry):
        pltpu.sync_copy(src_hbm.at[w * CPW + j], sbuf)
        pltpu.sync_copy(dst_hbm.at[w * CPW + j], dbuf)
        pltpu.sync_copy(tab_hbm.at[sbuf], rowsv)
        pltpu.sync_copy(rowsv, acc_sh.at[dbuf], add=True)
        return carry

    lax.fori_loop(0, CPW, _step, 0)
    plsc.subcore_barrier()
    pltpu.sync_copy(acc_sh.at[pl.ds(s * ROWS_PT, ROWS_PT)],
                    out_hbm.at[pl.ds(c * N_PAD + s * ROWS_PT, ROWS_PT)])


_agg = functools.partial(
    pl.kernel,
    mesh=_mesh,
    out_type=jax.ShapeDtypeStruct((NC * N_PAD, D), jnp.float32),
    scratch_types=[
        pltpu.VMEM((CH,), jnp.int32),
        pltpu.VMEM((CH,), jnp.int32),
        pltpu.VMEM((CH, D), jnp.float32),
        pltpu.VMEM_SHARED((N_PAD, D), jnp.float32),
    ],
)(_agg_body)


_RB = 1000


def _mm_body(x_ref, w_ref, o_ref):
    o_ref[...] = jnp.dot(x_ref[...], w_ref[...],
                         preferred_element_type=jnp.float32)


def _mm(x, w):
    return pl.pallas_call(
        _mm_body,
        grid=(N // _RB,),
        in_specs=[
            pl.BlockSpec((_RB, D), lambda i: (i, 0)),
            pl.BlockSpec((D, D), lambda i: (0, 0)),
        ],
        out_specs=pl.BlockSpec((_RB, D), lambda i: (i, 0)),
        out_shape=jax.ShapeDtypeStruct((N, D), jnp.float32),
    )(x, w)


def _scale_body(dp_ref, h_ref, xs_ref, dbc_ref):
    deg = dp_ref[0] + dp_ref[1]
    db = jnp.dot(deg, jnp.ones((16, D), jnp.float32),
                 preferred_element_type=jnp.float32) * (1.0 / 16.0)
    d = lax.rsqrt(db + 1.0)
    dbc_ref[...] = d
    xs_ref[...] = d * h_ref[...]


def _scale(degp, h):
    return pl.pallas_call(
        _scale_body,
        grid=(N // _RB,),
        in_specs=[
            pl.BlockSpec((NC, _RB, 16), lambda i: (0, i, 0)),
            pl.BlockSpec((_RB, D), lambda i: (i, 0)),
        ],
        out_specs=[
            pl.BlockSpec((_RB, D), lambda i: (i, 0)),
            pl.BlockSpec((_RB, D), lambda i: (i, 0)),
        ],
        out_shape=[
            jax.ShapeDtypeStruct((N, D), jnp.float32),
            jax.ShapeDtypeStruct((N, D), jnp.float32),
        ],
    )(degp, h)


def _comb_mm_body(p_ref, xs_ref, dbc_ref, b_ref, w_ref, o_ref):
    agg = p_ref[0] + p_ref[1] + xs_ref[...]
    t = jnp.maximum(dbc_ref[...] * agg + b_ref[...], 0.0)
    o_ref[...] = dbc_ref[...] * jnp.dot(t, w_ref[...],
                                        preferred_element_type=jnp.float32)


def _comb_mm(parts, xs, dbc, b, w):
    return pl.pallas_call(
        _comb_mm_body,
        grid=(N // _RB,),
        in_specs=[
            pl.BlockSpec((NC, _RB, D), lambda i: (0, i, 0)),
            pl.BlockSpec((_RB, D), lambda i: (i, 0)),
            pl.BlockSpec((_RB, D), lambda i: (i, 0)),
            pl.BlockSpec((1, D), lambda i: (0, 0)),
            pl.BlockSpec((D, D), lambda i: (0, 0)),
        ],
        out_specs=pl.BlockSpec((_RB, D), lambda i: (i, 0)),
        out_shape=jax.ShapeDtypeStruct((N, D), jnp.float32),
    )(parts, xs, dbc, b, w)


def _comb_body(p_ref, xs_ref, dbc_ref, b_ref, o_ref):
    agg = p_ref[0] + p_ref[1] + xs_ref[...]
    o_ref[...] = jnp.maximum(dbc_ref[...] * agg + b_ref[...], 0.0)


def _comb(parts, xs, dbc, b):
    return pl.pallas_call(
        _comb_body,
        grid=(N // _RB,),
        in_specs=[
            pl.BlockSpec((NC, _RB, D), lambda i: (0, i, 0)),
            pl.BlockSpec((_RB, D), lambda i: (i, 0)),
            pl.BlockSpec((_RB, D), lambda i: (i, 0)),
            pl.BlockSpec((1, D), lambda i: (0, 0)),
        ],
        out_specs=pl.BlockSpec((_RB, D), lambda i: (i, 0)),
        out_shape=jax.ShapeDtypeStruct((N, D), jnp.float32),
    )(parts, xs, dbc, b)


def kernel(x, edge_index, W1, b1, W2, b2):
    x = x.astype(jnp.float32)
    src = edge_index[0].astype(jnp.int32)
    dst = edge_index[1].astype(jnp.int32)
    pad = E_PAD - src.shape[0]
    srcp = jnp.concatenate(
        [src, jnp.zeros((pad,), jnp.int32)]).reshape(NW * CPW, CH)
    dstp = jnp.concatenate(
        [dst, jnp.full((pad,), TRASH, jnp.int32)]).reshape(NW * CPW, CH)

    z16 = jnp.zeros((N_PAD, 16), jnp.float32)
    zD = jnp.zeros((N_PAD, D), jnp.float32)
    degp = _deg(dstp, z16).reshape(NC, N_PAD, 16)
    h1 = _mm(x, W1)
    xs1, dbc = _scale(degp, h1)
    p1 = _agg(xs1, srcp, dstp, zD).reshape(NC, N_PAD, D)
    xs2 = _comb_mm(p1, xs1, dbc, b1.reshape(1, D), W2)
    p2 = _agg(xs2, srcp, dstp, zD).reshape(NC, N_PAD, D)
    return _comb(p2, xs2, dbc, b2.reshape(1, D))


# speedup vs baseline: 7.7877x; 1.0770x over previous
import functools

import jax
import jax.numpy as jnp
from jax import lax
from jax.experimental import pallas as pl
from jax.experimental.pallas import tpu as pltpu
from jax.experimental.pallas import tpu_sc as plsc

N = 10000
D = 128
E = 320000

NC = 2
NS = 16
NW = NC * NS
CH = 128
CPW = 80
EPW = CPW * CH
E_PAD = NW * EPW
ROWS_PT = 640
N_PAD = NS * ROWS_PT
TRASH = N

_mesh = plsc.VectorSubcoreMesh(core_axis_name="c", subcore_axis_name="s")


def _deg_body(dst_hbm, zeros_hbm, out_hbm, dbuf, onesv, deg_sh):
    c = lax.axis_index("c")
    s = lax.axis_index("s")
    w = s * NC + c

    pltpu.sync_copy(zeros_hbm.at[pl.ds(s * ROWS_PT, ROWS_PT)],
                    deg_sh.at[pl.ds(s * ROWS_PT, ROWS_PT)])

    def _orow(i, carry):
        onesv[i, :] = jnp.ones((16,), jnp.float32)
        return carry

    lax.fori_loop(0, CH, _orow, 0)
    plsc.subcore_barrier()

    def _step(j, carry):
        pltpu.sync_copy(dst_hbm.at[w * CPW + j], dbuf)
        pltpu.sync_copy(onesv, deg_sh.at[dbuf], add=True)
        return carry

    lax.fori_loop(0, CPW, _step, 0)
    plsc.subcore_barrier()
    pltpu.sync_copy(deg_sh.at[pl.ds(s * ROWS_PT, ROWS_PT)],
                    out_hbm.at[pl.ds(c * N_PAD + s * ROWS_PT, ROWS_PT)])


_deg = functools.partial(
    pl.kernel,
    mesh=_mesh,
    out_type=jax.ShapeDtypeStruct((NC * N_PAD, 16), jnp.float32),
    scratch_types=[
        pltpu.VMEM((CH,), jnp.int32),
        pltpu.VMEM((CH, 16), jnp.float32),
        pltpu.VMEM_SHARED((N_PAD, 16), jnp.float32),
    ],
)(_deg_body)


def _agg_body(tab_hbm, src_hbm, dst_hbm, zeros_hbm, out_hbm,
              s0, s1, d0, d1, rows0, rows1, acc_sh, semB):
    c = lax.axis_index("c")
    s = lax.axis_index("s")
    w = s * NC + c
    base = w * CPW

    pltpu.sync_copy(zeros_hbm.at[pl.ds(s * ROWS_PT, ROWS_PT)],
                    acc_sh.at[pl.ds(s * ROWS_PT, ROWS_PT)])
    plsc.subcore_barrier()

    pltpu.async_copy(src_hbm.at[base], s0, semB)
    pltpu.async_copy(dst_hbm.at[base], d0, semB)
    pltpu.async_copy(src_hbm.at[base + 1], s1, semB)
    pltpu.async_copy(dst_hbm.at[base + 1], d1, semB)
    pltpu.make_async_copy(src_hbm.at[base], s0, semB).wait()
    pltpu.make_async_copy(dst_hbm.at[base], d0, semB).wait()
    pltpu.make_async_copy(src_hbm.at[base + 1], s1, semB).wait()
    pltpu.make_async_copy(dst_hbm.at[base + 1], d1, semB).wait()

    def _pair(g, carry):
        j = base + 2 * g
        pltpu.sync_copy(tab_hbm.at[s0], rows0)
        pltpu.sync_copy(rows0, acc_sh.at[d0], add=True)
        pltpu.async_copy(src_hbm.at[j + 2], s0, semB)
        pltpu.async_copy(dst_hbm.at[j + 2], d0, semB)
        pltpu.sync_copy(tab_hbm.at[s1], rows1)
        pltpu.sync_copy(rows1, acc_sh.at[d1], add=True)
        pltpu.async_copy(src_hbm.at[j + 3], s1, semB)
        pltpu.async_copy(dst_hbm.at[j + 3], d1, semB)
        pltpu.make_async_copy(src_hbm.at[j + 2], s0, semB).wait()
        pltpu.make_async_copy(dst_hbm.at[j + 2], d0, semB).wait()
        pltpu.make_async_copy(src_hbm.at[j + 3], s1, semB).wait()
        pltpu.make_async_copy(dst_hbm.at[j + 3], d1, semB).wait()
        return carry

    lax.fori_loop(0, CPW // 2 - 1, _pair, 0)
    pltpu.sync_copy(tab_hbm.at[s0], rows0)
    pltpu.sync_copy(rows0, acc_sh.at[d0], add=True)
    pltpu.sync_copy(tab_hbm.at[s1], rows1)
    pltpu.sync_copy(rows1, acc_sh.at[d1], add=True)

    plsc.subcore_barrier()
    pltpu.sync_copy(acc_sh.at[pl.ds(s * ROWS_PT, ROWS_PT)],
                    out_hbm.at[pl.ds(c * N_PAD + s * ROWS_PT, ROWS_PT)])


_agg = functools.partial(
    pl.kernel,
    mesh=_mesh,
    out_type=jax.ShapeDtypeStruct((NC * N_PAD, D), jnp.float32),
    scratch_types=[
        pltpu.VMEM((CH,), jnp.int32),
        pltpu.VMEM((CH,), jnp.int32),
        pltpu.VMEM((CH,), jnp.int32),
        pltpu.VMEM((CH,), jnp.int32),
        pltpu.VMEM((CH, D), jnp.float32),
        pltpu.VMEM((CH, D), jnp.float32),
        pltpu.VMEM_SHARED((N_PAD, D), jnp.float32),
        pltpu.SemaphoreType.DMA,
    ],
)(_agg_body)


_RB = 1000


def _mm_body(x_ref, w_ref, o_ref):
    o_ref[...] = jnp.dot(x_ref[...], w_ref[...],
                         preferred_element_type=jnp.float32)


def _mm(x, w):
    return pl.pallas_call(
        _mm_body,
        grid=(N // _RB,),
        in_specs=[
            pl.BlockSpec((_RB, D), lambda i: (i, 0)),
            pl.BlockSpec((D, D), lambda i: (0, 0)),
        ],
        out_specs=pl.BlockSpec((_RB, D), lambda i: (i, 0)),
        out_shape=jax.ShapeDtypeStruct((N, D), jnp.float32),
    )(x, w)


def _scale_body(dp_ref, h_ref, xs_ref, dbc_ref):
    deg = dp_ref[0] + dp_ref[1]
    db = jnp.dot(deg, jnp.ones((16, D), jnp.float32),
                 preferred_element_type=jnp.float32) * (1.0 / 16.0)
    d = lax.rsqrt(db + 1.0)
    dbc_ref[...] = d
    xs_ref[...] = d * h_ref[...]


def _scale(degp, h):
    return pl.pallas_call(
        _scale_body,
        grid=(N // _RB,),
        in_specs=[
            pl.BlockSpec((NC, _RB, 16), lambda i: (0, i, 0)),
            pl.BlockSpec((_RB, D), lambda i: (i, 0)),
        ],
        out_specs=[
            pl.BlockSpec((_RB, D), lambda i: (i, 0)),
            pl.BlockSpec((_RB, D), lambda i: (i, 0)),
        ],
        out_shape=[
            jax.ShapeDtypeStruct((N, D), jnp.float32),
            jax.ShapeDtypeStruct((N, D), jnp.float32),
        ],
    )(degp, h)


def _comb_mm_body(p_ref, xs_ref, dbc_ref, b_ref, w_ref, o_ref):
    agg = p_ref[0] + p_ref[1] + xs_ref[...]
    t = jnp.maximum(dbc_ref[...] * agg + b_ref[...], 0.0)
    o_ref[...] = dbc_ref[...] * jnp.dot(t, w_ref[...],
                                        preferred_element_type=jnp.float32)


def _comb_mm(parts, xs, dbc, b, w):
    return pl.pallas_call(
        _comb_mm_body,
        grid=(N // _RB,),
        in_specs=[
            pl.BlockSpec((NC, _RB, D), lambda i: (0, i, 0)),
            pl.BlockSpec((_RB, D), lambda i: (i, 0)),
            pl.BlockSpec((_RB, D), lambda i: (i, 0)),
            pl.BlockSpec((1, D), lambda i: (0, 0)),
            pl.BlockSpec((D, D), lambda i: (0, 0)),
        ],
        out_specs=pl.BlockSpec((_RB, D), lambda i: (i, 0)),
        out_shape=jax.ShapeDtypeStruct((N, D), jnp.float32),
    )(parts, xs, dbc, b, w)


def _comb_body(p_ref, xs_ref, dbc_ref, b_ref, o_ref):
    agg = p_ref[0] + p_ref[1] + xs_ref[...]
    o_ref[...] = jnp.maximum(dbc_ref[...] * agg + b_ref[...], 0.0)


def _comb(parts, xs, dbc, b):
    return pl.pallas_call(
        _comb_body,
        grid=(N // _RB,),
        in_specs=[
            pl.BlockSpec((NC, _RB, D), lambda i: (0, i, 0)),
            pl.BlockSpec((_RB, D), lambda i: (i, 0)),
            pl.BlockSpec((_RB, D), lambda i: (i, 0)),
            pl.BlockSpec((1, D), lambda i: (0, 0)),
        ],
        out_specs=pl.BlockSpec((_RB, D), lambda i: (i, 0)),
        out_shape=jax.ShapeDtypeStruct((N, D), jnp.float32),
    )(parts, xs, dbc, b)


def kernel(x, edge_index, W1, b1, W2, b2):
    x = x.astype(jnp.float32)
    src = edge_index[0].astype(jnp.int32)
    dst = edge_index[1].astype(jnp.int32)
    pad = E_PAD - src.shape[0]
    srcp = jnp.concatenate(
        [src, jnp.zeros((pad,), jnp.int32)]).reshape(NW * CPW, CH)
    dstp = jnp.concatenate(
        [dst, jnp.full((pad,), TRASH, jnp.int32)]).reshape(NW * CPW, CH)

    z16 = jnp.zeros((N_PAD, 16), jnp.float32)
    zD = jnp.zeros((N_PAD, D), jnp.float32)
    degp = _deg(dstp, z16).reshape(NC, N_PAD, 16)
    h1 = _mm(x, W1)
    xs1, dbc = _scale(degp, h1)
    p1 = _agg(xs1, srcp, dstp, zD).reshape(NC, N_PAD, D)
    xs2 = _comb_mm(p1, xs1, dbc, b1.reshape(1, D), W2)
    p2 = _agg(xs2, srcp, dstp, zD).reshape(NC, N_PAD, D)
    return _comb(p2, xs2, dbc, b2.reshape(1, D))
